# baseline (speedup 1.0000x reference)
import functools
import jax
import jax.numpy as jnp
from jax import lax
from jax.experimental import pallas as pl
from jax.experimental.pallas import tpu as pltpu
from jax.experimental.pallas import tpu_sc as plsc

N = 10000
E = 320000
D = 128
G = 64
OUT = 10

NC = 2
NS = 16
NW = NC * NS
CHUNK = 128
NCHT = 2560
CPT = NCHT // NW
EPAD = NCHT * CHUNK - E
NP = 10016
JUNK = 10008
RPT = 624
RPT_LAST = N - RPT * (NS - 1)
ZR = 208

BLK = 1000
NB = N // BLK
NBP = NP // BLK + 1

_SC_MESH = plsc.VectorSubcoreMesh(core_axis_name="c", subcore_axis_name="s")


def _zero_acc(sid, zrows_hbm, acc):
    base = sid * RPT
    for k in range(RPT // ZR):
        pltpu.sync_copy(zrows_hbm, acc.at[pl.ds(base + k * ZR, ZR), :])

    @pl.when(sid == NS - 1)
    def _zero_tail():
        pltpu.sync_copy(zrows_hbm.at[pl.ds(0, RPT_LAST - RPT), :],
                        acc.at[pl.ds(N - (RPT_LAST - RPT), RPT_LAST - RPT), :])


def _writeout_acc(cid, sid, acc, out_hbm):
    @pl.when(sid < NS - 1)
    def _wr():
        pltpu.sync_copy(acc.at[pl.ds(sid * RPT, RPT), :],
                        out_hbm.at[cid, pl.ds(sid * RPT, RPT), :])

    @pl.when(sid == NS - 1)
    def _wr_last():
        base = RPT * (NS - 1)
        pltpu.sync_copy(acc.at[pl.ds(base, RPT_LAST), :],
                        out_hbm.at[cid, pl.ds(base, RPT_LAST), :])


EPT_W = CPT * CHUNK


def _decode(epkall, j, idxs, idxd):
    for q in range(CHUNK // 16):
        w = epkall[pl.ds(j * CHUNK + q * 16, 16)]
        if idxs is not None:
            idxs[pl.ds(q * 16, 16)] = w & 0xFFFF
        if idxd is not None:
            idxd[pl.ds(q * 16, 16)] = lax.shift_right_logical(w, 16)


@functools.partial(
    pl.kernel,
    out_type=jax.ShapeDtypeStruct((NC, N, D), jnp.float32),
    mesh=_SC_MESH,
    scratch_types=[
        pltpu.VMEM((EPT_W,), jnp.int32),
        pltpu.VMEM((CHUNK,), jnp.int32),
        pltpu.VMEM((CHUNK, D), jnp.float32),
        pltpu.VMEM_SHARED((N, D), jnp.float32),
    ],
)
def _sc_degree(epk_hbm, ones_hbm, zrows_hbm, degp_hbm,
               epkall, idxd, ones_v, deg_t):
    cid = lax.axis_index("c")
    sid = lax.axis_index("s")
    wid = sid * NC + cid

    pltpu.sync_copy(ones_hbm, ones_v)
    _zero_acc(sid, zrows_hbm, deg_t)
    pltpu.sync_copy(epk_hbm.at[pl.ds(wid * EPT_W, EPT_W)], epkall)
    plsc.subcore_barrier()

    def body(j, carry):
        _decode(epkall, j, None, idxd)
        pltpu.sync_copy(ones_v, deg_t.at[idxd], add=True)
        return carry

    lax.fori_loop(0, CPT, body, 0)

    plsc.subcore_barrier()
    _writeout_acc(cid, sid, deg_t, degp_hbm)


@functools.partial(
    pl.kernel,
    out_type=jax.ShapeDtypeStruct((NC, N, D), jnp.float32),
    mesh=_SC_MESH,
    scratch_types=[
        pltpu.VMEM((EPT_W,), jnp.int32),
        pltpu.VMEM((CHUNK,), jnp.int32),
        pltpu.VMEM((CHUNK,), jnp.int32),
        pltpu.VMEM((CHUNK,), jnp.int32),
        pltpu.VMEM((CHUNK,), jnp.int32),
        pltpu.VMEM((CHUNK, D), jnp.float32),
        pltpu.VMEM((CHUNK, D), jnp.float32),
        pltpu.VMEM_SHARED((N, D), jnp.float32),
        pltpu.SemaphoreType.DMA,
        pltpu.SemaphoreType.DMA,
    ],
)
def _sc_message(hws_hbm, epk_hbm, zrows_hbm, accp_hbm,
                epkall, idxs0, idxd0, idxs1, idxd1, rows0, rows1,
                acc, s0, s1):
    cid = lax.axis_index("c")
    sid = lax.axis_index("s")
    wid = sid * NC + cid

    _zero_acc(sid, zrows_hbm, acc)
    pltpu.sync_copy(epk_hbm.at[pl.ds(wid * EPT_W, EPT_W)], epkall)
    plsc.subcore_barrier()

    _decode(epkall, 0, idxs0, idxd0)
    pltpu.async_copy(hws_hbm.at[idxs0], rows0, s0)

    def body(jj, carry):
        j0 = jj * 2
        _decode(epkall, j0 + 1, idxs1, idxd1)
        d1 = pltpu.async_copy(hws_hbm.at[idxs1], rows1, s1)
        pltpu.make_async_copy(zrows_hbm.at[pl.ds(0, CHUNK), :],
                              rows0, s0).wait()
        pltpu.sync_copy(rows0, acc.at[idxd0], add=True)

        @pl.when(jj < CPT // 2 - 1)
        def _next():
            _decode(epkall, j0 + 2, idxs0, idxd0)
            pltpu.async_copy(hws_hbm.at[idxs0], rows0, s0)

        d1.wait()
        pltpu.sync_copy(rows1, acc.at[idxd1], add=True)
        return carry

    lax.fori_loop(0, CPT // 2, body, 0)

    plsc.subcore_barrier()
    _writeout_acc(cid, sid, acc, accp_hbm)


_HI = lax.Precision.HIGHEST


def _t0_body(deg_ref, x_ref, w_ref, dis_ref, hws_ref):
    i = pl.program_id(0)
    rowid = i * BLK + lax.broadcasted_iota(jnp.int32, (BLK, 1), 0)
    valid = rowid < N
    d = jnp.sum(deg_ref[...], axis=1, keepdims=True) + 1.0
    d = d - jnp.where(rowid < EPAD, 1.0, 0.0)
    dis = jnp.where(valid, lax.rsqrt(d), 0.0)
    dis_ref[...] = dis
    hws_ref[...] = dis * jnp.dot(x_ref[...], w_ref[...],
                                 preferred_element_type=jnp.float32,
                                 precision=_HI)


def _tc_first(degT, x, W0):
    return pl.pallas_call(
        _t0_body,
        grid=(NBP,),
        in_specs=[
            pl.BlockSpec((BLK, NC), lambda i: (i, 0)),
            pl.BlockSpec((BLK, D), lambda i: (i, 0)),
            pl.BlockSpec((D, D), lambda i: (0, 0)),
        ],
        out_specs=[
            pl.BlockSpec((BLK, 1), lambda i: (i, 0)),
            pl.BlockSpec((BLK, D), lambda i: (i, 0)),
        ],
        out_shape=[
            jax.ShapeDtypeStruct((NP, 1), jnp.float32),
            jax.ShapeDtypeStruct((NP, D), jnp.float32),
        ],
    )(degT, x, W0)


def _tl_body(accp_ref, hws_ref, dis_ref, b_ref, w_ref, out_ref):
    i = pl.program_id(0)
    rowid = i * BLK + lax.broadcasted_iota(jnp.int32, (BLK, 1), 0)
    valid = rowid < N
    acc = accp_ref[0] + accp_ref[1]
    dis = dis_ref[...]
    h = jnp.maximum(dis * (acc + hws_ref[...]) + b_ref[...], 0.0)
    out_ref[...] = jnp.where(
        valid,
        dis * jnp.dot(h, w_ref[...], preferred_element_type=jnp.float32,
                      precision=_HI),
        0.0)


def _tc_layer(accp, hws, dis, b, Wn):
    return pl.pallas_call(
        _tl_body,
        grid=(NBP,),
        in_specs=[
            pl.BlockSpec((NC, BLK, D), lambda i: (0, i, 0)),
            pl.BlockSpec((BLK, D), lambda i: (i, 0)),
            pl.BlockSpec((BLK, 1), lambda i: (i, 0)),
            pl.BlockSpec((1, D), lambda i: (0, 0)),
            pl.BlockSpec((D, D), lambda i: (0, 0)),
        ],
        out_specs=pl.BlockSpec((BLK, D), lambda i: (i, 0)),
        out_shape=jax.ShapeDtypeStruct((NP, D), jnp.float32),
    )(accp, hws, dis, b, Wn)


def _t3_body(accp_ref, hws_ref, dis_ref, b_ref, br_ref, w_ref, bo_ref,
             out_ref, gmax, gsum, cnt):
    step = pl.program_id(0)

    @pl.when(step == 0)
    def _init():
        gmax[...] = jnp.full((G, D), -jnp.inf, jnp.float32)
        gsum[...] = jnp.zeros((G, D), jnp.float32)
        cnt[...] = jnp.zeros((G, D), jnp.float32)

    acc = accp_ref[0] + accp_ref[1]
    dis = dis_ref[...]
    h = jnp.maximum(dis * (acc + hws_ref[...]) + b_ref[...], 0.0)

    br = br_ref[...]
    onehot = (br == lax.broadcasted_iota(jnp.int32, (BLK, G), 1)
              ).astype(jnp.float32)
    dgen = (((0,), (0,)), ((), ()))
    gsum[...] += lax.dot_general(onehot, h, dgen,
                                 preferred_element_type=jnp.float32,
                                 precision=_HI)
    cnt[...] += lax.dot_general(onehot, jnp.ones((BLK, D), jnp.float32),
                                dgen, preferred_element_type=jnp.float32,
                                precision=_HI)

    def gbody(g, carry):
        m = br == g
        v = jnp.max(jnp.where(m, h, -jnp.inf), axis=0, keepdims=True)
        gmax[pl.ds(g, 1), :] = jnp.maximum(gmax[pl.ds(g, 1), :], v)
        return carry

    lax.fori_loop(0, G, gbody, 0)

    @pl.when(step == NB - 1)
    def _head():
        c = cnt[...]
        gmaxf = jnp.where(c > 0, gmax[...], 0.0)
        gs = gsum[...]
        gmean = gs / jnp.maximum(c, 1.0)
        w = w_ref[...]
        out_ref[...] = (
            jnp.dot(gmaxf, w[0:D, :], preferred_element_type=jnp.float32,
                    precision=_HI)
            + jnp.dot(gmean, w[D:2 * D, :],
                      preferred_element_type=jnp.float32, precision=_HI)
            + jnp.dot(gs, w[2 * D:3 * D, :],
                      preferred_element_type=jnp.float32, precision=_HI)
            + bo_ref[...])


def _tc_pool_head(accp, hws, dis, b, br, Wp, bop):
    return pl.pallas_call(
        _t3_body,
        grid=(NB,),
        in_specs=[
            pl.BlockSpec((NC, BLK, D), lambda i: (0, i, 0)),
            pl.BlockSpec((BLK, D), lambda i: (i, 0)),
            pl.BlockSpec((BLK, 1), lambda i: (i, 0)),
            pl.BlockSpec((1, D), lambda i: (0, 0)),
            pl.BlockSpec((BLK, 1), lambda i: (i, 0)),
            pl.BlockSpec((3 * D, D), lambda i: (0, 0)),
            pl.BlockSpec((1, D), lambda i: (0, 0)),
        ],
        out_specs=pl.BlockSpec((G, D), lambda i: (0, 0)),
        out_shape=jax.ShapeDtypeStruct((G, D), jnp.float32),
        scratch_shapes=[
            pltpu.VMEM((G, D), jnp.float32),
            pltpu.VMEM((G, D), jnp.float32),
            pltpu.VMEM((G, D), jnp.float32),
        ],
    )(accp, hws, dis, b, br, Wp, bop)


@jax.jit
def kernel(x, edge_index, batch, W0, b0, W1, b1, W2, b2, Wout, bout):
    src = edge_index[0]
    dst = edge_index[1]
    srcp = jnp.concatenate([src, jnp.full((EPAD,), JUNK, jnp.int32)])
    dstp = jnp.concatenate([dst, jnp.arange(EPAD, dtype=jnp.int32)])
    epk = jnp.bitwise_or(srcp, dstp << 16)

    zrows = jnp.zeros((ZR, D), jnp.float32)
    ones_c = jnp.ones((CHUNK, D), jnp.float32)

    degp = _sc_degree(epk, ones_c, zrows)
    degT = degp[:, :, 0].T

    dis, hws = _tc_first(degT, x, W0)

    b0r = b0.reshape(1, D)
    b1r = b1.reshape(1, D)
    b2r = b2.reshape(1, D)
    br = batch.reshape(N, 1)
    Wp = jnp.pad(Wout, ((0, 0), (0, D - OUT)))
    bop = jnp.pad(bout, (0, D - OUT)).reshape(1, D)

    accp = _sc_message(hws, epk, zrows)
    hws1 = _tc_layer(accp, hws, dis, b0r, W1)
    accp = _sc_message(hws1, epk, zrows)
    hws2 = _tc_layer(accp, hws1, dis, b1r, W2)
    accp = _sc_message(hws2, epk, zrows)
    out128 = _tc_pool_head(accp, hws2, dis, b2r, br, Wp, bop)
    return out128[:, :OUT]


# speedup vs baseline: 2.4481x; 2.4481x over previous
import functools
import jax
import jax.numpy as jnp
from jax import lax
from jax.experimental import pallas as pl
from jax.experimental.pallas import tpu as pltpu
from jax.experimental.pallas import tpu_sc as plsc

N = 10000
E = 320000
D = 128
G = 64
OUT = 10

NC = 2
NS = 16
NW = NC * NS
CHUNK = 128
NCHT = 2560
CPT = NCHT // NW
EPAD = NCHT * CHUNK - E
NP = 10016
JUNK = 10008
RPT = 624
RPT_LAST = N - RPT * (NS - 1)
ZR = 208

BLK = 1000
NB = N // BLK
NBP = NP // BLK + 1

_SC_MESH = plsc.VectorSubcoreMesh(core_axis_name="c", subcore_axis_name="s")

import numpy as _np
_CHUNK_PERM = _np.concatenate([_np.arange(w, NCHT, NW) for w in range(NW)])


def _zero_acc(sid, zrows_hbm, acc):
    base = sid * RPT
    for k in range(RPT // ZR):
        pltpu.sync_copy(zrows_hbm, acc.at[pl.ds(base + k * ZR, ZR), :])

    @pl.when(sid == NS - 1)
    def _zero_tail():
        pltpu.sync_copy(zrows_hbm.at[pl.ds(0, RPT_LAST - RPT), :],
                        acc.at[pl.ds(N - (RPT_LAST - RPT), RPT_LAST - RPT), :])


def _writeout_acc(cid, sid, acc, out_hbm):
    @pl.when(sid < NS - 1)
    def _wr():
        pltpu.sync_copy(acc.at[pl.ds(sid * RPT, RPT), :],
                        out_hbm.at[cid, pl.ds(sid * RPT, RPT), :])

    @pl.when(sid == NS - 1)
    def _wr_last():
        base = RPT * (NS - 1)
        pltpu.sync_copy(acc.at[pl.ds(base, RPT_LAST), :],
                        out_hbm.at[cid, pl.ds(base, RPT_LAST), :])


EPT_W = CPT * CHUNK


def _decode(epkall, j, idxs, idxd):
    for q in range(CHUNK // 16):
        w = epkall[pl.ds(j * CHUNK + q * 16, 16)]
        if idxs is not None:
            idxs[pl.ds(q * 16, 16)] = w & 0xFFFF
        if idxd is not None:
            idxd[pl.ds(q * 16, 16)] = lax.shift_right_logical(w, 16)


@functools.partial(
    pl.kernel,
    out_type=jax.ShapeDtypeStruct((NC, N, D), jnp.float32),
    mesh=_SC_MESH,
    scratch_types=[
        pltpu.VMEM((EPT_W,), jnp.int32),
        pltpu.VMEM((CHUNK,), jnp.int32),
        pltpu.VMEM((CHUNK, D), jnp.float32),
        pltpu.VMEM_SHARED((N, D), jnp.float32),
    ],
)
def _sc_degree(epk_hbm, ones_hbm, zrows_hbm, degp_hbm,
               epkall, idxd, ones_v, deg_t):
    cid = lax.axis_index("c")
    sid = lax.axis_index("s")
    wid = sid * NC + cid

    pltpu.sync_copy(ones_hbm, ones_v)
    _zero_acc(sid, zrows_hbm, deg_t)
    pltpu.sync_copy(epk_hbm.at[pl.ds(wid * EPT_W, EPT_W)], epkall)
    plsc.subcore_barrier()

    def body(j, carry):
        _decode(epkall, j, None, idxd)
        pltpu.sync_copy(ones_v, deg_t.at[idxd], add=True)
        return carry

    lax.fori_loop(0, CPT, body, 0)

    plsc.subcore_barrier()
    _writeout_acc(cid, sid, deg_t, degp_hbm)


@functools.partial(
    pl.kernel,
    out_type=jax.ShapeDtypeStruct((NC, N, D), jnp.float32),
    mesh=_SC_MESH,
    scratch_types=[
        pltpu.VMEM((EPT_W,), jnp.int32),
        pltpu.VMEM((CHUNK,), jnp.int32),
        pltpu.VMEM((CHUNK,), jnp.int32),
        pltpu.VMEM((CHUNK,), jnp.int32),
        pltpu.VMEM((CHUNK,), jnp.int32),
        pltpu.VMEM((CHUNK, D), jnp.float32),
        pltpu.VMEM((CHUNK, D), jnp.float32),
        pltpu.VMEM_SHARED((N, D), jnp.float32),
        pltpu.SemaphoreType.DMA,
        pltpu.SemaphoreType.DMA,
    ],
)
def _sc_message(hws_hbm, epk_hbm, zrows_hbm, accp_hbm,
                epkall, idxs0, idxd0, idxs1, idxd1, rows0, rows1,
                acc, s0, s1):
    cid = lax.axis_index("c")
    sid = lax.axis_index("s")
    wid = sid * NC + cid

    _zero_acc(sid, zrows_hbm, acc)
    pltpu.sync_copy(epk_hbm.at[pl.ds(wid * EPT_W, EPT_W)], epkall)
    plsc.subcore_barrier()

    _decode(epkall, 0, idxs0, idxd0)
    pltpu.async_copy(hws_hbm.at[idxs0], rows0, s0)

    def body(jj, carry):
        j0 = jj * 2
        _decode(epkall, j0 + 1, idxs1, idxd1)
        d1 = pltpu.async_copy(hws_hbm.at[idxs1], rows1, s1)
        pltpu.make_async_copy(zrows_hbm.at[pl.ds(0, CHUNK), :],
                              rows0, s0).wait()
        pltpu.sync_copy(rows0, acc.at[idxd0], add=True)

        @pl.when(jj < CPT // 2 - 1)
        def _next():
            _decode(epkall, j0 + 2, idxs0, idxd0)
            pltpu.async_copy(hws_hbm.at[idxs0], rows0, s0)

        d1.wait()
        pltpu.sync_copy(rows1, acc.at[idxd1], add=True)
        return carry

    lax.fori_loop(0, CPT // 2, body, 0)

    plsc.subcore_barrier()
    _writeout_acc(cid, sid, acc, accp_hbm)


_HI = lax.Precision.HIGHEST


def _t0_body(deg_ref, x_ref, w_ref, dis_ref, hws_ref):
    i = pl.program_id(0)
    rowid = i * BLK + lax.broadcasted_iota(jnp.int32, (BLK, 1), 0)
    valid = rowid < N
    d = jnp.sum(deg_ref[...], axis=1, keepdims=True) + 1.0
    d = d - jnp.where(rowid < EPAD, 1.0, 0.0)
    dis = jnp.where(valid, lax.rsqrt(d), 0.0)
    dis_ref[...] = dis
    hws_ref[...] = dis * jnp.dot(x_ref[...], w_ref[...],
                                 preferred_element_type=jnp.float32,
                                 precision=_HI)


def _tc_first(degT, x, W0):
    return pl.pallas_call(
        _t0_body,
        grid=(NBP,),
        in_specs=[
            pl.BlockSpec((BLK, NC), lambda i: (i, 0)),
            pl.BlockSpec((BLK, D), lambda i: (i, 0)),
            pl.BlockSpec((D, D), lambda i: (0, 0)),
        ],
        out_specs=[
            pl.BlockSpec((BLK, 1), lambda i: (i, 0)),
            pl.BlockSpec((BLK, D), lambda i: (i, 0)),
        ],
        out_shape=[
            jax.ShapeDtypeStruct((NP, 1), jnp.float32),
            jax.ShapeDtypeStruct((NP, D), jnp.float32),
        ],
    )(degT, x, W0)


def _tl_body(accp_ref, hws_ref, dis_ref, b_ref, w_ref, out_ref):
    i = pl.program_id(0)
    rowid = i * BLK + lax.broadcasted_iota(jnp.int32, (BLK, 1), 0)
    valid = rowid < N
    acc = accp_ref[0] + accp_ref[1]
    dis = dis_ref[...]
    h = jnp.maximum(dis * (acc + hws_ref[...]) + b_ref[...], 0.0)
    out_ref[...] = jnp.where(
        valid,
        dis * jnp.dot(h, w_ref[...], preferred_element_type=jnp.float32,
                      precision=_HI),
        0.0)


def _tc_layer(accp, hws, dis, b, Wn):
    return pl.pallas_call(
        _tl_body,
        grid=(NBP,),
        in_specs=[
            pl.BlockSpec((NC, BLK, D), lambda i: (0, i, 0)),
            pl.BlockSpec((BLK, D), lambda i: (i, 0)),
            pl.BlockSpec((BLK, 1), lambda i: (i, 0)),
            pl.BlockSpec((1, D), lambda i: (0, 0)),
            pl.BlockSpec((D, D), lambda i: (0, 0)),
        ],
        out_specs=pl.BlockSpec((BLK, D), lambda i: (i, 0)),
        out_shape=jax.ShapeDtypeStruct((NP, D), jnp.float32),
    )(accp, hws, dis, b, Wn)


def _t3_body(accp_ref, hws_ref, dis_ref, b_ref, br_ref, w_ref, bo_ref,
             out_ref, gmax, gsum, cnt):
    step = pl.program_id(0)

    @pl.when(step == 0)
    def _init():
        gmax[...] = jnp.full((G, D), -jnp.inf, jnp.float32)
        gsum[...] = jnp.zeros((G, D), jnp.float32)
        cnt[...] = jnp.zeros((G, D), jnp.float32)

    acc = accp_ref[0] + accp_ref[1]
    dis = dis_ref[...]
    h = jnp.maximum(dis * (acc + hws_ref[...]) + b_ref[...], 0.0)

    br = br_ref[...]
    onehot = (br == lax.broadcasted_iota(jnp.int32, (BLK, G), 1)
              ).astype(jnp.float32)
    dgen = (((0,), (0,)), ((), ()))
    gsum[...] += lax.dot_general(onehot, h, dgen,
                                 preferred_element_type=jnp.float32,
                                 precision=_HI)
    cnt[...] += lax.dot_general(onehot, jnp.ones((BLK, D), jnp.float32),
                                dgen, preferred_element_type=jnp.float32,
                                precision=_HI)

    def gbody(g, carry):
        m = br == g
        v = jnp.max(jnp.where(m, h, -jnp.inf), axis=0, keepdims=True)
        gmax[pl.ds(g, 1), :] = jnp.maximum(gmax[pl.ds(g, 1), :], v)
        return carry

    lax.fori_loop(0, G, gbody, 0)

    @pl.when(step == NB - 1)
    def _head():
        c = cnt[...]
        gmaxf = jnp.where(c > 0, gmax[...], 0.0)
        gs = gsum[...]
        gmean = gs / jnp.maximum(c, 1.0)
        w = w_ref[...]
        out_ref[...] = (
            jnp.dot(gmaxf, w[0:D, :], preferred_element_type=jnp.float32,
                    precision=_HI)
            + jnp.dot(gmean, w[D:2 * D, :],
                      preferred_element_type=jnp.float32, precision=_HI)
            + jnp.dot(gs, w[2 * D:3 * D, :],
                      preferred_element_type=jnp.float32, precision=_HI)
            + bo_ref[...])


def _tc_pool_head(accp, hws, dis, b, br, Wp, bop):
    return pl.pallas_call(
        _t3_body,
        grid=(NB,),
        in_specs=[
            pl.BlockSpec((NC, BLK, D), lambda i: (0, i, 0)),
            pl.BlockSpec((BLK, D), lambda i: (i, 0)),
            pl.BlockSpec((BLK, 1), lambda i: (i, 0)),
            pl.BlockSpec((1, D), lambda i: (0, 0)),
            pl.BlockSpec((BLK, 1), lambda i: (i, 0)),
            pl.BlockSpec((3 * D, D), lambda i: (0, 0)),
            pl.BlockSpec((1, D), lambda i: (0, 0)),
        ],
        out_specs=pl.BlockSpec((G, D), lambda i: (0, 0)),
        out_shape=jax.ShapeDtypeStruct((G, D), jnp.float32),
        scratch_shapes=[
            pltpu.VMEM((G, D), jnp.float32),
            pltpu.VMEM((G, D), jnp.float32),
            pltpu.VMEM((G, D), jnp.float32),
        ],
    )(accp, hws, dis, b, br, Wp, bop)


@jax.jit
def kernel(x, edge_index, batch, W0, b0, W1, b1, W2, b2, Wout, bout):
    src = edge_index[0]
    dst = edge_index[1]
    srcp = jnp.concatenate(
        [src, N + jnp.arange(EPAD, dtype=jnp.int32) % (NP - N)])
    dstp = jnp.concatenate([dst, jnp.arange(EPAD, dtype=jnp.int32)])
    epk = jnp.bitwise_or(srcp, dstp << 16)
    epk = epk.reshape(NCHT, CHUNK)[_CHUNK_PERM].reshape(-1)

    zrows = jnp.zeros((ZR, D), jnp.float32)
    ones_c = jnp.ones((CHUNK, D), jnp.float32)

    degp = _sc_degree(epk, ones_c, zrows)
    degT = degp[:, :, 0].T

    dis, hws = _tc_first(degT, x, W0)

    b0r = b0.reshape(1, D)
    b1r = b1.reshape(1, D)
    b2r = b2.reshape(1, D)
    br = batch.reshape(N, 1)
    Wp = jnp.pad(Wout, ((0, 0), (0, D - OUT)))
    bop = jnp.pad(bout, (0, D - OUT)).reshape(1, D)

    accp = _sc_message(hws, epk, zrows)
    hws1 = _tc_layer(accp, hws, dis, b0r, W1)
    accp = _sc_message(hws1, epk, zrows)
    hws2 = _tc_layer(accp, hws1, dis, b1r, W2)
    accp = _sc_message(hws2, epk, zrows)
    out128 = _tc_pool_head(accp, hws2, dis, b2r, br, Wp, bop)
    return out128[:, :OUT]
